# initial kernel scaffold (unmeasured)
import jax
import jax.numpy as jnp
from jax import lax
from jax.experimental import pallas as pl
from jax.experimental.pallas import tpu as pltpu


def kernel(
    x,
):
    def body(*refs):
        pass

    out_shape = jax.ShapeDtypeStruct(..., jnp.float32)
    return pl.pallas_call(body, out_shape=out_shape)(...)


# baseline (device time: 32373 ns/iter reference)
import jax
import jax.numpy as jnp
from jax import lax
from jax.experimental import pallas as pl
from jax.experimental.pallas import tpu as pltpu

M = 2048
N = 1024
N_HALF = N // 2


def kernel(x):
    def body(x_ref, out_ref, send_buf, recv_buf, send_sem, recv_sem):
        my_x = lax.axis_index("x")
        my_y = lax.axis_index("y")
        my_z = lax.axis_index("z")
        other_x = 1 - my_x

        barrier_sem = pltpu.get_barrier_semaphore()
        pl.semaphore_signal(
            barrier_sem,
            inc=1,
            device_id=(other_x, my_y, my_z),
            device_id_type=pl.DeviceIdType.MESH,
        )
        pl.semaphore_wait(barrier_sem, 1)

        send_buf[:, :] = x_ref[0, :, pl.ds(other_x * N_HALF, N_HALF)].astype(
            jnp.bfloat16
        )

        rdma = pltpu.make_async_remote_copy(
            src_ref=send_buf,
            dst_ref=recv_buf,
            send_sem=send_sem,
            recv_sem=recv_sem,
            device_id=(other_x, my_y, my_z),
            device_id_type=pl.DeviceIdType.MESH,
        )
        rdma.start()
        rdma.wait()

        out_ref[:, :] = x_ref[0, :, pl.ds(my_x * N_HALF, N_HALF)] + recv_buf[
            :, :
        ].astype(jnp.float32)

    return pl.pallas_call(
        body,
        out_shape=jax.ShapeDtypeStruct((M, N_HALF), jnp.float32),
        in_specs=[pl.BlockSpec(memory_space=pltpu.VMEM)],
        out_specs=pl.BlockSpec(memory_space=pltpu.VMEM),
        scratch_shapes=[
            pltpu.VMEM((M, N_HALF), jnp.bfloat16),
            pltpu.VMEM((M, N_HALF), jnp.bfloat16),
            pltpu.SemaphoreType.DMA,
            pltpu.SemaphoreType.DMA,
        ],
        compiler_params=pltpu.CompilerParams(collective_id=0),
    )(x)


# device time: 24679 ns/iter; 1.3118x vs baseline; 1.3118x over previous
import jax
import jax.numpy as jnp
from jax import lax
from jax.experimental import pallas as pl
from jax.experimental.pallas import tpu as pltpu

M = 2048
N = 1024
N_HALF = N // 2
M_HALF = M // 2
K = 8
ROWS = M_HALF // K


def kernel(x):
    def body(x_ref, out_ref, stage, rx, ry, sx_send, sx_recv, sy_send, sy_recv):
        my_x = lax.axis_index("x")
        my_y = lax.axis_index("y")
        my_z = lax.axis_index("z")
        px = (1 - my_x, my_y, my_z)
        qy = (my_x, 1 - my_y, my_z)

        barrier_sem = pltpu.get_barrier_semaphore()
        for nbr in (px, qy):
            pl.semaphore_signal(
                barrier_sem,
                inc=1,
                device_id=nbr,
                device_id_type=pl.DeviceIdType.MESH,
            )
        pl.semaphore_wait(barrier_sem, 2)

        half_row0 = my_y * M_HALF
        other_row0 = (1 - my_y) * M_HALF
        my_c0 = my_x * N_HALF
        other_c0 = (1 - my_x) * N_HALF

        x_rdmas = []
        for c in range(K):
            stage[c, :, :] = x_ref[
                0, pl.ds(half_row0 + c * ROWS, ROWS), pl.ds(other_c0, N_HALF)
            ].astype(jnp.bfloat16)
            r = pltpu.make_async_remote_copy(
                src_ref=stage.at[c],
                dst_ref=rx.at[c],
                send_sem=sx_send.at[c],
                recv_sem=sx_recv.at[c],
                device_id=px,
                device_id_type=pl.DeviceIdType.MESH,
            )
            r.start()
            x_rdmas.append(r)

        y_rdmas = []
        for c in range(K):
            x_rdmas[c].wait_recv()
            r = pltpu.make_async_remote_copy(
                src_ref=rx.at[c],
                dst_ref=ry.at[c],
                send_sem=sy_send.at[c],
                recv_sem=sy_recv.at[c],
                device_id=qy,
                device_id_type=pl.DeviceIdType.MESH,
            )
            r.start()
            y_rdmas.append(r)
            out_ref[pl.ds(half_row0 + c * ROWS, ROWS), :] = (
                x_ref[0, pl.ds(half_row0 + c * ROWS, ROWS), pl.ds(my_c0, N_HALF)]
                + rx[c, :, :].astype(jnp.float32)
            )

        for c in range(K):
            y_rdmas[c].wait_recv()
            out_ref[pl.ds(other_row0 + c * ROWS, ROWS), :] = (
                x_ref[0, pl.ds(other_row0 + c * ROWS, ROWS), pl.ds(my_c0, N_HALF)]
                + ry[c, :, :].astype(jnp.float32)
            )

        for c in range(K):
            x_rdmas[c].wait_send()
            y_rdmas[c].wait_send()

    return pl.pallas_call(
        body,
        out_shape=jax.ShapeDtypeStruct((M, N_HALF), jnp.float32),
        in_specs=[pl.BlockSpec(memory_space=pltpu.VMEM)],
        out_specs=pl.BlockSpec(memory_space=pltpu.VMEM),
        scratch_shapes=[
            pltpu.VMEM((K, ROWS, N_HALF), jnp.bfloat16),
            pltpu.VMEM((K, ROWS, N_HALF), jnp.bfloat16),
            pltpu.VMEM((K, ROWS, N_HALF), jnp.bfloat16),
            pltpu.SemaphoreType.DMA((K,)),
            pltpu.SemaphoreType.DMA((K,)),
            pltpu.SemaphoreType.DMA((K,)),
            pltpu.SemaphoreType.DMA((K,)),
        ],
        compiler_params=pltpu.CompilerParams(collective_id=0),
    )(x)
